# baseline (speedup 1.0000x reference)
import jax
import jax.numpy as jnp
from jax.experimental import pallas as pl
from jax.experimental.pallas import tpu as pltpu

_CACHE_SIZE = 4096.0 + 16.0
_NUM_KV_HEADS = 8
_NUM_BLOCKS = 2048
_BS = 64
_C = 512
_GH = 1


def _rotate_body(sb_ref, pos_ref, x_ref, o_ref):
    i = pl.program_id(1)
    bid = jax.lax.broadcasted_iota(jnp.int32, (_BS, _C), 1) + i * _C
    barange = jax.lax.broadcasted_iota(jnp.int32, (_BS, _C), 0)
    match = bid == sb_ref[...]
    key = jnp.where(match, barange, -1)
    w = jnp.max(key, axis=0, keepdims=True)
    onehot = jnp.logical_and(barange == w, match)
    posf = pos_ref[...].astype(jnp.float32)
    ev = jnp.maximum(posf - _CACHE_SIZE, 0.0)
    theta = jnp.sum(jnp.where(onehot, ev, 0.0), axis=0, keepdims=True)

    d8i = jax.lax.broadcasted_iota(jnp.int32, (1, 8, 1, 8, _C), 1)
    li = jax.lax.broadcasted_iota(jnp.int32, (1, 8, 1, 8, _C), 3)
    j = (d8i * 8 + li).astype(jnp.float32)
    inv_freq = jnp.exp(j * (-jnp.log(10000.0) / 64.0))
    freqs = theta.reshape(1, 1, 1, 1, _C) * inv_freq
    c = jnp.cos(freqs)
    s = jnp.sin(freqs)

    x = x_ref[...]
    x1 = x[:, :8]
    x2 = x[:, 8:]
    o_ref[:, :8] = x1 * c - x2 * s
    o_ref[:, 8:] = x2 * c + x1 * s


def kernel(key_cache, block_tables, positions):
    x = jnp.transpose(key_cache, (1, 2, 3, 4, 0))
    sb = block_tables[:, :1]
    pos = positions.reshape(_BS, 1)
    out = pl.pallas_call(
        _rotate_body,
        grid=(_NUM_KV_HEADS // _GH, _NUM_BLOCKS // _C),
        in_specs=[
            pl.BlockSpec((_BS, 1), lambda h, i: (0, 0)),
            pl.BlockSpec((_BS, 1), lambda h, i: (0, 0)),
            pl.BlockSpec((_GH, 16, 16, 8, _C), lambda h, i: (h, 0, 0, 0, i)),
        ],
        out_specs=pl.BlockSpec((_GH, 16, 16, 8, _C), lambda h, i: (h, 0, 0, 0, i)),
        out_shape=jax.ShapeDtypeStruct((_NUM_KV_HEADS, 16, 16, 8, _NUM_BLOCKS), jnp.float32),
        compiler_params=pltpu.CompilerParams(
            dimension_semantics=("arbitrary", "arbitrary"),
        ),
    )(sb, pos, x)
    return jnp.transpose(out, (4, 0, 1, 2, 3))


# speedup vs baseline: 1.0598x; 1.0598x over previous
import jax
import jax.numpy as jnp
from jax.experimental import pallas as pl
from jax.experimental.pallas import tpu as pltpu

_CACHE_SIZE = 4096.0 + 16.0
_NUM_KV_HEADS = 8
_NUM_BLOCKS = 2048
_BS = 64
_C = 128
_GH = 8


def _rotate_body(sb_ref, pos_ref, x_ref, o_ref):
    i = pl.program_id(1)
    bid = jax.lax.broadcasted_iota(jnp.int32, (_BS, _C), 1) + i * _C
    barange = jax.lax.broadcasted_iota(jnp.int32, (_BS, _C), 0)
    match = bid == sb_ref[...]
    key = jnp.where(match, barange, -1)
    w = jnp.max(key, axis=0, keepdims=True)
    onehot = jnp.logical_and(barange == w, match)
    posf = pos_ref[...].astype(jnp.float32)
    ev = jnp.maximum(posf - _CACHE_SIZE, 0.0)
    theta = jnp.sum(jnp.where(onehot, ev, 0.0), axis=0, keepdims=True)

    d8i = jax.lax.broadcasted_iota(jnp.int32, (1, 8, 1, 8, _C), 1)
    li = jax.lax.broadcasted_iota(jnp.int32, (1, 8, 1, 8, _C), 3)
    j = (d8i * 8 + li).astype(jnp.float32)
    inv_freq = jnp.exp(j * (-jnp.log(10000.0) / 64.0))
    freqs = theta.reshape(1, 1, 1, 1, _C) * inv_freq
    c = jnp.cos(freqs)
    s = jnp.sin(freqs)

    for h in range(_GH):
        x1 = x_ref[h, :8]
        x2 = x_ref[h, 8:]
        o_ref[h, :8] = x1 * c[0] - x2 * s[0]
        o_ref[h, 8:] = x2 * c[0] + x1 * s[0]


def kernel(key_cache, block_tables, positions):
    x = jnp.transpose(key_cache, (1, 2, 3, 4, 0))
    sb = block_tables[:, :1]
    pos = positions.reshape(_BS, 1)
    out = pl.pallas_call(
        _rotate_body,
        grid=(_NUM_KV_HEADS // _GH, _NUM_BLOCKS // _C),
        in_specs=[
            pl.BlockSpec((_BS, 1), lambda h, i: (0, 0)),
            pl.BlockSpec((_BS, 1), lambda h, i: (0, 0)),
            pl.BlockSpec((_GH, 16, 16, 8, _C), lambda h, i: (h, 0, 0, 0, i)),
        ],
        out_specs=pl.BlockSpec((_GH, 16, 16, 8, _C), lambda h, i: (h, 0, 0, 0, i)),
        out_shape=jax.ShapeDtypeStruct((_NUM_KV_HEADS, 16, 16, 8, _NUM_BLOCKS), jnp.float32),
        compiler_params=pltpu.CompilerParams(
            dimension_semantics=("arbitrary", "arbitrary"),
        ),
    )(sb, pos, x)
    return jnp.transpose(out, (4, 0, 1, 2, 3))
